# baseline (speedup 1.0000x reference)
import functools

import jax
import jax.numpy as jnp
from jax import lax
from jax.experimental import pallas as pl
from jax.experimental.pallas import tpu as pltpu
from jax.experimental.pallas import tpu_sc as plsc

_EPS = 1e-09
_GATHER_ROWS = 128
_SLABS_PER_BLOCK = 16


def _sc_gather(token_table, idx2d, seq):
    d = token_table.shape[1]
    n = idx2d.shape[0]
    num_workers = 32
    per_w = n // num_workers
    half = per_w // 2
    steps = half // _GATHER_ROWS
    mesh = plsc.VectorSubcoreMesh(core_axis_name="c", subcore_axis_name="s")

    @functools.partial(
        pl.kernel,
        out_type=jax.ShapeDtypeStruct((n // 2, 2 * d), token_table.dtype),
        mesh=mesh,
        scratch_types=[
            pltpu.VMEM((per_w,), jnp.float32),
            pltpu.VMEM((per_w,), jnp.int32),
            pltpu.VMEM((_GATHER_ROWS, d), token_table.dtype),
            pltpu.VMEM((_GATHER_ROWS, d), token_table.dtype),
            pltpu.SemaphoreType.DMA,
            pltpu.SemaphoreType.DMA,
            pltpu.SemaphoreType.DMA,
        ],
        compiler_params=pltpu.CompilerParams(
            use_tc_tiling_on_sc=False, needs_layout_passes=False),
    )
    def gather_kernel(table_hbm, idx_hbm, out_hbm, idxf_v, idx_v,
                      rows_a, rows_b, sem_a, sem_b, sem_o):
        wid = lax.axis_index("s") * 2 + lax.axis_index("c")
        out_row0 = wid * half
        pltpu.sync_copy(idx_hbm.at[pl.ds(wid * per_w, per_w)], idxf_v)

        @pl.loop(0, per_w // 16)
        def _(i):
            v = idxf_v[pl.ds(i * 16, 16)]
            idx_v[pl.ds(i * 16, 16)] = plsc.bitcast(v, jnp.int32)

        @pl.loop(0, steps)
        def _(k):
            ga = pltpu.async_copy(
                table_hbm.at[idx_v.at[pl.ds(k * _GATHER_ROWS,
                                            _GATHER_ROWS)]],
                rows_a, sem_a)
            gb = pltpu.async_copy(
                table_hbm.at[idx_v.at[pl.ds(half + k * _GATHER_ROWS,
                                            _GATHER_ROWS)]],
                rows_b, sem_b)
            out_rows = pl.ds(out_row0 + k * _GATHER_ROWS, _GATHER_ROWS)
            ga.wait()
            oa = pltpu.async_copy(
                rows_a, out_hbm.at[out_rows, pl.ds(0, d)], sem_o)
            gb.wait()
            ob = pltpu.async_copy(
                rows_b, out_hbm.at[out_rows, pl.ds(d, d)], sem_o)
            oa.wait()
            ob.wait()

    return gather_kernel(token_table, idx2d)


def _flatten_body(group, in_ref, out_ref):
    n_rows, seq = in_ref.shape
    for g in range(0, n_rows, group):
        flat = jnp.concatenate([in_ref[g + j, :] for j in range(group)])
        out_ref[pl.ds(g * seq, group * seq)] = flat


def _tc_flatten(x2d):
    batch, seq = x2d.shape
    block = 512
    group = 16
    return pl.pallas_call(
        functools.partial(_flatten_body, group),
        grid=(batch // block,),
        in_specs=[pl.BlockSpec((block, seq), lambda i: (i, 0))],
        out_specs=pl.BlockSpec((block * seq,), lambda i: (i,)),
        out_shape=jax.ShapeDtypeStruct((batch * seq,), jnp.int32),
    )(x2d)


def _ln_body(seq, d, h_ref, pos_ref, b_ref, out_ref):
    bvec = b_ref[...]
    sign = jnp.where(
        lax.broadcasted_iota(jnp.int32, (seq, 2 * d), 1) < d, 1.0, -1.0
    )
    for p in range(_SLABS_PER_BLOCK):
        r0 = p * seq
        h = h_ref[pl.ds(r0, seq), :] + pos_ref[...]
        hs = h * sign
        hh = h * h
        hhs = hs * h
        s_all = jnp.sum(h, axis=-1, keepdims=True)
        s_sgn = jnp.sum(hs, axis=-1, keepdims=True)
        q_all = jnp.sum(hh, axis=-1, keepdims=True)
        q_sgn = jnp.sum(hhs, axis=-1, keepdims=True)
        mean = (s_all + sign * s_sgn) * (0.5 / d)
        qh = (q_all + sign * q_sgn) * 0.5
        var_sum = qh - mean * mean * d
        scale = lax.rsqrt(var_sum * (1.0 / (d - 1)) + _EPS * _EPS)
        out = (h - mean) * scale + bvec
        out_ref[0, pl.ds(r0, seq), :] = out[:, :d]
        out_ref[1, pl.ds(r0, seq), :] = out[:, d:]


def _tc_layernorm(h_pair, pos_pair, b_pair, batch, seq, d):
    block_rows = seq * _SLABS_PER_BLOCK
    n_half = h_pair.shape[0]
    grid = n_half // block_rows
    half_span = 4 * block_rows
    return pl.pallas_call(
        functools.partial(_ln_body, seq, d),
        grid=(grid,),
        in_specs=[
            pl.BlockSpec((block_rows, 2 * d), lambda i: (i, 0)),
            pl.BlockSpec((seq, 2 * d), lambda i: (0, 0)),
            pl.BlockSpec((1, 2 * d), lambda i: (0, 0)),
        ],
        out_specs=pl.BlockSpec(
            (2, block_rows, d), lambda i: (i // 4, i % 4, 0)
        ),
        out_shape=jax.ShapeDtypeStruct(
            (2 * n_half // half_span, half_span, d), jnp.float32),
    )(h_pair, pos_pair, b_pair)


def kernel(x, token_table, pos_table, a, b):
    batch, seq = x.shape
    d = token_table.shape[1]
    n = batch * seq
    idx2d = lax.bitcast_convert_type(x.astype(jnp.int32),
                                     jnp.float32).reshape(-1)
    gathered = _sc_gather(token_table, idx2d, seq)
    pos = pos_table[:seq]
    pos_pair = jnp.concatenate([pos, pos], axis=1)
    b_pair = jnp.concatenate([b, b]).reshape(1, 2 * d)
    out = _tc_layernorm(gathered, pos_pair, b_pair, batch, seq, d)
    return out.reshape(batch, seq, d)


# speedup vs baseline: 1.0255x; 1.0255x over previous
import functools

import jax
import jax.numpy as jnp
from jax import lax
from jax.experimental import pallas as pl
from jax.experimental.pallas import tpu as pltpu
from jax.experimental.pallas import tpu_sc as plsc

_EPS = 1e-09
_GATHER_ROWS = 256
_SLABS_PER_BLOCK = 16


def _sc_gather(token_table, idx2d, seq):
    d = token_table.shape[1]
    n = idx2d.shape[0]
    num_workers = 32
    per_w = n // num_workers
    half = per_w // 2
    steps = half // _GATHER_ROWS
    mesh = plsc.VectorSubcoreMesh(core_axis_name="c", subcore_axis_name="s")

    @functools.partial(
        pl.kernel,
        out_type=jax.ShapeDtypeStruct((n // 2, 2 * d), token_table.dtype),
        mesh=mesh,
        scratch_types=[
            pltpu.VMEM((per_w,), jnp.int32),
            pltpu.VMEM((_GATHER_ROWS, d), token_table.dtype),
            pltpu.VMEM((_GATHER_ROWS, d), token_table.dtype),
            pltpu.SemaphoreType.DMA,
            pltpu.SemaphoreType.DMA,
            pltpu.SemaphoreType.DMA,
        ],
        compiler_params=pltpu.CompilerParams(use_tc_tiling_on_sc=False),
    )
    def gather_kernel(table_hbm, idx_hbm, out_hbm, idx_v,
                      rows_a, rows_b, sem_a, sem_b, sem_o):
        wid = lax.axis_index("s") * 2 + lax.axis_index("c")
        out_row0 = wid * half
        pltpu.sync_copy(idx_hbm.at[pl.ds(wid * per_w, per_w)], idx_v)

        @pl.loop(0, steps)
        def _(k):
            ga = pltpu.async_copy(
                table_hbm.at[idx_v.at[pl.ds(k * _GATHER_ROWS,
                                            _GATHER_ROWS)]],
                rows_a, sem_a)
            gb = pltpu.async_copy(
                table_hbm.at[idx_v.at[pl.ds(half + k * _GATHER_ROWS,
                                            _GATHER_ROWS)]],
                rows_b, sem_b)
            out_rows = pl.ds(out_row0 + k * _GATHER_ROWS, _GATHER_ROWS)
            ga.wait()
            oa = pltpu.async_copy(
                rows_a, out_hbm.at[out_rows, pl.ds(0, d)], sem_o)
            gb.wait()
            ob = pltpu.async_copy(
                rows_b, out_hbm.at[out_rows, pl.ds(d, d)], sem_o)
            oa.wait()
            ob.wait()

    return gather_kernel(token_table, idx2d)


def _flatten_body(group, in_ref, out_ref):
    n_rows, seq = in_ref.shape
    for g in range(0, n_rows, group):
        flat = jnp.concatenate([in_ref[g + j, :] for j in range(group)])
        out_ref[pl.ds(g * seq, group * seq)] = flat


def _tc_flatten(x2d):
    batch, seq = x2d.shape
    block = 512
    group = 16
    return pl.pallas_call(
        functools.partial(_flatten_body, group),
        grid=(batch // block,),
        in_specs=[pl.BlockSpec((block, seq), lambda i: (i, 0))],
        out_specs=pl.BlockSpec((block * seq,), lambda i: (i,)),
        out_shape=jax.ShapeDtypeStruct((batch * seq,), jnp.int32),
    )(x2d)


def _ln_body(seq, d, h_ref, pos_ref, b_ref, out_ref):
    bvec = b_ref[...]
    sign = jnp.where(
        lax.broadcasted_iota(jnp.int32, (seq, 2 * d), 1) < d, 1.0, -1.0
    )
    for p in range(_SLABS_PER_BLOCK):
        r0 = p * seq
        h = h_ref[pl.ds(r0, seq), :] + pos_ref[...]
        hs = h * sign
        hh = h * h
        hhs = hs * h
        s_all = jnp.sum(h, axis=-1, keepdims=True)
        s_sgn = jnp.sum(hs, axis=-1, keepdims=True)
        q_all = jnp.sum(hh, axis=-1, keepdims=True)
        q_sgn = jnp.sum(hhs, axis=-1, keepdims=True)
        mean = (s_all + sign * s_sgn) * (0.5 / d)
        qh = (q_all + sign * q_sgn) * 0.5
        var_sum = qh - mean * mean * d
        scale = lax.rsqrt(var_sum * (1.0 / (d - 1)) + _EPS * _EPS)
        out = (h - mean) * scale + bvec
        out_ref[0, pl.ds(r0, seq), :] = out[:, :d]
        out_ref[1, pl.ds(r0, seq), :] = out[:, d:]


def _tc_layernorm(h_pair, pos_pair, b_pair, batch, seq, d):
    block_rows = seq * _SLABS_PER_BLOCK
    n_half = h_pair.shape[0]
    grid = n_half // block_rows
    half_span = 4 * block_rows
    return pl.pallas_call(
        functools.partial(_ln_body, seq, d),
        grid=(grid,),
        in_specs=[
            pl.BlockSpec((block_rows, 2 * d), lambda i: (i, 0)),
            pl.BlockSpec((seq, 2 * d), lambda i: (0, 0)),
            pl.BlockSpec((1, 2 * d), lambda i: (0, 0)),
        ],
        out_specs=pl.BlockSpec(
            (2, block_rows, d), lambda i: (i // 4, i % 4, 0)
        ),
        out_shape=jax.ShapeDtypeStruct(
            (2 * n_half // half_span, half_span, d), jnp.float32),
    )(h_pair, pos_pair, b_pair)


def kernel(x, token_table, pos_table, a, b):
    batch, seq = x.shape
    d = token_table.shape[1]
    n = batch * seq
    idx2d = x.astype(jnp.int32).reshape(-1)
    gathered = _sc_gather(token_table, idx2d, seq)
    pos = pos_table[:seq]
    pos_pair = jnp.concatenate([pos, pos], axis=1)
    b_pair = jnp.concatenate([b, b]).reshape(1, 2 * d)
    out = _tc_layernorm(gathered, pos_pair, b_pair, batch, seq, d)
    return out.reshape(batch, seq, d)


# speedup vs baseline: 1.0361x; 1.0103x over previous
import functools

import jax
import jax.numpy as jnp
from jax import lax
from jax.experimental import pallas as pl
from jax.experimental.pallas import tpu as pltpu
from jax.experimental.pallas import tpu_sc as plsc

_EPS = 1e-09
_GATHER_ROWS = 320
_SLABS_PER_BLOCK = 16


def _sc_gather(token_table, idx2d, seq):
    d = token_table.shape[1]
    n = idx2d.shape[0]
    num_workers = 32
    per_w = n // num_workers
    half = per_w // 2
    steps = half // _GATHER_ROWS
    mesh = plsc.VectorSubcoreMesh(core_axis_name="c", subcore_axis_name="s")

    @functools.partial(
        pl.kernel,
        out_type=jax.ShapeDtypeStruct((n // 2, 2 * d), token_table.dtype),
        mesh=mesh,
        scratch_types=[
            pltpu.VMEM((per_w,), jnp.int32),
            pltpu.VMEM((_GATHER_ROWS, d), token_table.dtype),
            pltpu.VMEM((_GATHER_ROWS, d), token_table.dtype),
            pltpu.VMEM((_GATHER_ROWS, d), token_table.dtype),
            pltpu.VMEM((_GATHER_ROWS, d), token_table.dtype),
            pltpu.SemaphoreType.DMA,
            pltpu.SemaphoreType.DMA,
            pltpu.SemaphoreType.DMA,
            pltpu.SemaphoreType.DMA,
            pltpu.SemaphoreType.DMA,
            pltpu.SemaphoreType.DMA,
        ],
        compiler_params=pltpu.CompilerParams(use_tc_tiling_on_sc=False),
    )
    def gather_kernel(table_hbm, idx_hbm, out_hbm, idx_v,
                      rows_ap, rows_bp, rows_aq, rows_bq,
                      sga_p, sgb_p, sga_q, sgb_q, swo_p, swo_q):
        wid = lax.axis_index("s") * 2 + lax.axis_index("c")
        out_row0 = wid * half
        g = _GATHER_ROWS
        pltpu.sync_copy(idx_hbm.at[pl.ds(wid * per_w, per_w)], idx_v)

        def gath(k, ra, rb, sa, sb):
            pltpu.async_copy(table_hbm.at[idx_v.at[pl.ds(k * g, g)]],
                             ra, sa)
            pltpu.async_copy(table_hbm.at[idx_v.at[pl.ds(half + k * g, g)]],
                             rb, sb)

        def gath_wait(k, ra, rb, sa, sb):
            pltpu.make_async_copy(
                table_hbm.at[idx_v.at[pl.ds(k * g, g)]], ra, sa).wait()
            pltpu.make_async_copy(
                table_hbm.at[idx_v.at[pl.ds(half + k * g, g)]],
                rb, sb).wait()

        def wb(k, ra, rb, so):
            rows = pl.ds(out_row0 + k * g, g)
            pltpu.async_copy(ra, out_hbm.at[rows, pl.ds(0, d)], so)
            pltpu.async_copy(rb, out_hbm.at[rows, pl.ds(d, d)], so)

        def wb_wait(k, ra, rb, so):
            rows = pl.ds(out_row0 + k * g, g)
            pltpu.make_async_copy(
                ra, out_hbm.at[rows, pl.ds(0, d)], so).wait()
            pltpu.make_async_copy(
                rb, out_hbm.at[rows, pl.ds(d, d)], so).wait()

        st2 = steps // 2
        gath(0, rows_ap, rows_bp, sga_p, sgb_p)

        @pl.loop(0, st2)
        def _(i):
            k0 = 2 * i
            k1 = k0 + 1

            @pl.when(i > 0)
            def _():
                wb_wait(k0 - 1, rows_aq, rows_bq, swo_q)

            gath(k1, rows_aq, rows_bq, sga_q, sgb_q)
            gath_wait(k0, rows_ap, rows_bp, sga_p, sgb_p)
            wb(k0, rows_ap, rows_bp, swo_p)
            gath_wait(k1, rows_aq, rows_bq, sga_q, sgb_q)
            wb(k1, rows_aq, rows_bq, swo_q)
            wb_wait(k0, rows_ap, rows_bp, swo_p)

            @pl.when(i < st2 - 1)
            def _():
                gath(k0 + 2, rows_ap, rows_bp, sga_p, sgb_p)

        wb_wait(steps - 1, rows_aq, rows_bq, swo_q)

    return gather_kernel(token_table, idx2d)


def _flatten_body(group, in_ref, out_ref):
    n_rows, seq = in_ref.shape
    for g in range(0, n_rows, group):
        flat = jnp.concatenate([in_ref[g + j, :] for j in range(group)])
        out_ref[pl.ds(g * seq, group * seq)] = flat


def _tc_flatten(x2d):
    batch, seq = x2d.shape
    block = 512
    group = 16
    return pl.pallas_call(
        functools.partial(_flatten_body, group),
        grid=(batch // block,),
        in_specs=[pl.BlockSpec((block, seq), lambda i: (i, 0))],
        out_specs=pl.BlockSpec((block * seq,), lambda i: (i,)),
        out_shape=jax.ShapeDtypeStruct((batch * seq,), jnp.int32),
    )(x2d)


def _ln_body(seq, d, h_ref, pos_ref, b_ref, out_ref):
    bvec = b_ref[...]
    sign = jnp.where(
        lax.broadcasted_iota(jnp.int32, (seq, 2 * d), 1) < d, 1.0, -1.0
    )
    for p in range(_SLABS_PER_BLOCK):
        r0 = p * seq
        h = h_ref[pl.ds(r0, seq), :] + pos_ref[...]
        hs = h * sign
        hh = h * h
        hhs = hs * h
        s_all = jnp.sum(h, axis=-1, keepdims=True)
        s_sgn = jnp.sum(hs, axis=-1, keepdims=True)
        q_all = jnp.sum(hh, axis=-1, keepdims=True)
        q_sgn = jnp.sum(hhs, axis=-1, keepdims=True)
        mean = (s_all + sign * s_sgn) * (0.5 / d)
        qh = (q_all + sign * q_sgn) * 0.5
        var_sum = qh - mean * mean * d
        scale = lax.rsqrt(var_sum * (1.0 / (d - 1)) + _EPS * _EPS)
        out = (h - mean) * scale + bvec
        out_ref[0, pl.ds(r0, seq), :] = out[:, :d]
        out_ref[1, pl.ds(r0, seq), :] = out[:, d:]


def _tc_layernorm(h_pair, pos_pair, b_pair, batch, seq, d):
    block_rows = seq * _SLABS_PER_BLOCK
    n_half = h_pair.shape[0]
    grid = n_half // block_rows
    half_span = 4 * block_rows
    return pl.pallas_call(
        functools.partial(_ln_body, seq, d),
        grid=(grid,),
        in_specs=[
            pl.BlockSpec((block_rows, 2 * d), lambda i: (i, 0)),
            pl.BlockSpec((seq, 2 * d), lambda i: (0, 0)),
            pl.BlockSpec((1, 2 * d), lambda i: (0, 0)),
        ],
        out_specs=pl.BlockSpec(
            (2, block_rows, d), lambda i: (i // 4, i % 4, 0)
        ),
        out_shape=jax.ShapeDtypeStruct(
            (2 * n_half // half_span, half_span, d), jnp.float32),
    )(h_pair, pos_pair, b_pair)


def kernel(x, token_table, pos_table, a, b):
    batch, seq = x.shape
    d = token_table.shape[1]
    n = batch * seq
    idx2d = x.astype(jnp.int32).reshape(-1)
    gathered = _sc_gather(token_table, idx2d, seq)
    pos = pos_table[:seq]
    pos_pair = jnp.concatenate([pos, pos], axis=1)
    b_pair = jnp.concatenate([b, b]).reshape(1, 2 * d)
    out = _tc_layernorm(gathered, pos_pair, b_pair, batch, seq, d)
    return out.reshape(batch, seq, d)


# speedup vs baseline: 1.1494x; 1.1094x over previous
import functools

import jax
import jax.numpy as jnp
from jax import lax
from jax.experimental import pallas as pl
from jax.experimental.pallas import tpu as pltpu
from jax.experimental.pallas import tpu_sc as plsc

_EPS = 1e-09
_GATHER_ROWS = 320
_SLABS_PER_BLOCK = 16


def _sc_gather(token_table, idx2d, seq):
    d = token_table.shape[1]
    n = idx2d.shape[0]
    num_workers = 32
    per_w = n // num_workers
    half = per_w // 2
    steps = half // _GATHER_ROWS
    mesh = plsc.VectorSubcoreMesh(core_axis_name="c", subcore_axis_name="s")

    @functools.partial(
        pl.kernel,
        out_type=jax.ShapeDtypeStruct((n // 2, 2 * d), token_table.dtype),
        mesh=mesh,
        scratch_types=[
            pltpu.VMEM((per_w,), jnp.int32),
            pltpu.VMEM((_GATHER_ROWS, d), token_table.dtype),
            pltpu.VMEM((_GATHER_ROWS, d), token_table.dtype),
            pltpu.VMEM((_GATHER_ROWS, d), token_table.dtype),
            pltpu.VMEM((_GATHER_ROWS, d), token_table.dtype),
            pltpu.SemaphoreType.DMA,
            pltpu.SemaphoreType.DMA,
            pltpu.SemaphoreType.DMA,
            pltpu.SemaphoreType.DMA,
            pltpu.SemaphoreType.DMA,
            pltpu.SemaphoreType.DMA,
        ],
        compiler_params=pltpu.CompilerParams(use_tc_tiling_on_sc=False),
    )
    def gather_kernel(table_hbm, idx_hbm, out_hbm, idx_v,
                      rows_ap, rows_bp, rows_aq, rows_bq,
                      sga_p, sgb_p, sga_q, sgb_q, swo_p, swo_q):
        wid = lax.axis_index("s") * 2 + lax.axis_index("c")
        out_row0 = wid * half
        g = _GATHER_ROWS
        pltpu.sync_copy(idx_hbm.at[pl.ds(wid * per_w, per_w)], idx_v)

        def gath(k, ra, rb, sa, sb):
            pltpu.async_copy(table_hbm.at[idx_v.at[pl.ds(k * g, g)]],
                             ra, sa)
            pltpu.async_copy(table_hbm.at[idx_v.at[pl.ds(half + k * g, g)]],
                             rb, sb)

        def gath_wait(k, ra, rb, sa, sb):
            pltpu.make_async_copy(
                table_hbm.at[idx_v.at[pl.ds(k * g, g)]], ra, sa).wait()
            pltpu.make_async_copy(
                table_hbm.at[idx_v.at[pl.ds(half + k * g, g)]],
                rb, sb).wait()

        def wb(k, ra, rb, so):
            rows = pl.ds(out_row0 + k * g, g)
            pltpu.async_copy(ra, out_hbm.at[rows, pl.ds(0, d)], so)
            pltpu.async_copy(rb, out_hbm.at[rows, pl.ds(d, d)], so)

        def wb_wait(k, ra, rb, so):
            rows = pl.ds(out_row0 + k * g, g)
            pltpu.make_async_copy(
                ra, out_hbm.at[rows, pl.ds(0, d)], so).wait()
            pltpu.make_async_copy(
                rb, out_hbm.at[rows, pl.ds(d, d)], so).wait()

        st2 = steps // 2
        gath(0, rows_ap, rows_bp, sga_p, sgb_p)

        @pl.loop(0, st2)
        def _(i):
            k0 = 2 * i
            k1 = k0 + 1

            @pl.when(i > 0)
            def _():
                wb_wait(k0 - 1, rows_aq, rows_bq, swo_q)

            gath(k1, rows_aq, rows_bq, sga_q, sgb_q)
            gath_wait(k0, rows_ap, rows_bp, sga_p, sgb_p)
            wb(k0, rows_ap, rows_bp, swo_p)
            gath_wait(k1, rows_aq, rows_bq, sga_q, sgb_q)
            wb(k1, rows_aq, rows_bq, swo_q)
            wb_wait(k0, rows_ap, rows_bp, swo_p)

            @pl.when(i < st2 - 1)
            def _():
                gath(k0 + 2, rows_ap, rows_bp, sga_p, sgb_p)

        wb_wait(steps - 1, rows_aq, rows_bq, swo_q)

    return gather_kernel(token_table, idx2d)


def _flatten_body(group, in_ref, out_ref):
    n_rows, seq = in_ref.shape
    for g in range(0, n_rows, group):
        flat = jnp.concatenate([in_ref[g + j, :] for j in range(group)])
        out_ref[pl.ds(g * seq, group * seq)] = flat


def _tc_flatten(x2d):
    batch, seq = x2d.shape
    block = 512
    group = 16
    return pl.pallas_call(
        functools.partial(_flatten_body, group),
        grid=(batch // block,),
        in_specs=[pl.BlockSpec((block, seq), lambda i: (i, 0))],
        out_specs=pl.BlockSpec((block * seq,), lambda i: (i,)),
        out_shape=jax.ShapeDtypeStruct((batch * seq,), jnp.int32),
    )(x2d)


def _ln_body(seq, d, h_ref, pos_ref, b_ref, out_ref):
    rows = pos_ref.shape[0]
    bvec = b_ref[...]
    sign = jnp.where(
        lax.broadcasted_iota(jnp.int32, (rows, 2 * d), 1) < d, 1.0, -1.0
    )
    for p in range(_SLABS_PER_BLOCK * seq // rows):
        r0 = p * rows
        h = h_ref[pl.ds(r0, rows), :] + pos_ref[...]
        hs = h * sign
        hh = h * h
        hhs = hs * h
        s_all = jnp.sum(h, axis=-1, keepdims=True)
        s_sgn = jnp.sum(hs, axis=-1, keepdims=True)
        q_all = jnp.sum(hh, axis=-1, keepdims=True)
        q_sgn = jnp.sum(hhs, axis=-1, keepdims=True)
        mean = (s_all + sign * s_sgn) * (0.5 / d)
        qh = (q_all + sign * q_sgn) * 0.5
        var_sum = qh - mean * mean * d
        scale = lax.rsqrt(var_sum * (1.0 / (d - 1)) + _EPS * _EPS)
        out = (h - mean) * scale + bvec
        out_ref[0, pl.ds(r0, rows), :] = out[:, :d]
        out_ref[1, pl.ds(r0, rows), :] = out[:, d:]


def _tc_layernorm(h_pair, pos_pair, b_pair, batch, seq, d):
    block_rows = seq * _SLABS_PER_BLOCK
    n_half = h_pair.shape[0]
    grid = n_half // block_rows
    half_span = 4 * block_rows
    slab = pos_pair.shape[0]
    return pl.pallas_call(
        functools.partial(_ln_body, seq, d),
        grid=(grid,),
        in_specs=[
            pl.BlockSpec((block_rows, 2 * d), lambda i: (i, 0)),
            pl.BlockSpec((slab, 2 * d), lambda i: (0, 0)),
            pl.BlockSpec((1, 2 * d), lambda i: (0, 0)),
        ],
        out_specs=pl.BlockSpec(
            (2, block_rows, d), lambda i: (i // 4, i % 4, 0)
        ),
        out_shape=jax.ShapeDtypeStruct(
            (2 * n_half // half_span, half_span, d), jnp.float32),
    )(h_pair, pos_pair, b_pair)


def kernel(x, token_table, pos_table, a, b):
    batch, seq = x.shape
    d = token_table.shape[1]
    n = batch * seq
    idx2d = x.astype(jnp.int32).reshape(-1)
    gathered = _sc_gather(token_table, idx2d, seq)
    pos = pos_table[:seq]
    pos_pair = jnp.tile(jnp.concatenate([pos, pos], axis=1), (4, 1))
    b_pair = jnp.concatenate([b, b]).reshape(1, 2 * d)
    out = _tc_layernorm(gathered, pos_pair, b_pair, batch, seq, d)
    return out.reshape(batch, seq, d)


# speedup vs baseline: 1.1581x; 1.0076x over previous
import functools

import jax
import jax.numpy as jnp
from jax import lax
from jax.experimental import pallas as pl
from jax.experimental.pallas import tpu as pltpu
from jax.experimental.pallas import tpu_sc as plsc

_EPS = 1e-09
_GATHER_ROWS = 320
_SLABS_PER_BLOCK = 32


def _sc_gather(token_table, idx2d, seq):
    d = token_table.shape[1]
    n = idx2d.shape[0]
    num_workers = 32
    per_w = n // num_workers
    half = per_w // 2
    steps = half // _GATHER_ROWS
    mesh = plsc.VectorSubcoreMesh(core_axis_name="c", subcore_axis_name="s")

    @functools.partial(
        pl.kernel,
        out_type=jax.ShapeDtypeStruct((n // 2, 2 * d), token_table.dtype),
        mesh=mesh,
        scratch_types=[
            pltpu.VMEM((per_w,), jnp.int32),
            pltpu.VMEM((_GATHER_ROWS, d), token_table.dtype),
            pltpu.VMEM((_GATHER_ROWS, d), token_table.dtype),
            pltpu.VMEM((_GATHER_ROWS, d), token_table.dtype),
            pltpu.VMEM((_GATHER_ROWS, d), token_table.dtype),
            pltpu.SemaphoreType.DMA,
            pltpu.SemaphoreType.DMA,
            pltpu.SemaphoreType.DMA,
            pltpu.SemaphoreType.DMA,
            pltpu.SemaphoreType.DMA,
            pltpu.SemaphoreType.DMA,
        ],
        compiler_params=pltpu.CompilerParams(use_tc_tiling_on_sc=False),
    )
    def gather_kernel(table_hbm, idx_hbm, out_hbm, idx_v,
                      rows_ap, rows_bp, rows_aq, rows_bq,
                      sga_p, sgb_p, sga_q, sgb_q, swo_p, swo_q):
        wid = lax.axis_index("s") * 2 + lax.axis_index("c")
        out_row0 = wid * half
        g = _GATHER_ROWS
        pltpu.sync_copy(idx_hbm.at[pl.ds(wid * per_w, per_w)], idx_v)

        def gath(k, ra, rb, sa, sb):
            pltpu.async_copy(table_hbm.at[idx_v.at[pl.ds(k * g, g)]],
                             ra, sa)
            pltpu.async_copy(table_hbm.at[idx_v.at[pl.ds(half + k * g, g)]],
                             rb, sb)

        def gath_wait(k, ra, rb, sa, sb):
            pltpu.make_async_copy(
                table_hbm.at[idx_v.at[pl.ds(k * g, g)]], ra, sa).wait()
            pltpu.make_async_copy(
                table_hbm.at[idx_v.at[pl.ds(half + k * g, g)]],
                rb, sb).wait()

        def wb(k, ra, rb, so):
            rows = pl.ds(out_row0 + k * g, g)
            pltpu.async_copy(ra, out_hbm.at[rows, pl.ds(0, d)], so)
            pltpu.async_copy(rb, out_hbm.at[rows, pl.ds(d, d)], so)

        def wb_wait(k, ra, rb, so):
            rows = pl.ds(out_row0 + k * g, g)
            pltpu.make_async_copy(
                ra, out_hbm.at[rows, pl.ds(0, d)], so).wait()
            pltpu.make_async_copy(
                rb, out_hbm.at[rows, pl.ds(d, d)], so).wait()

        st2 = steps // 2
        gath(0, rows_ap, rows_bp, sga_p, sgb_p)

        @pl.loop(0, st2)
        def _(i):
            k0 = 2 * i
            k1 = k0 + 1

            @pl.when(i > 0)
            def _():
                wb_wait(k0 - 1, rows_aq, rows_bq, swo_q)

            gath(k1, rows_aq, rows_bq, sga_q, sgb_q)
            gath_wait(k0, rows_ap, rows_bp, sga_p, sgb_p)
            wb(k0, rows_ap, rows_bp, swo_p)
            gath_wait(k1, rows_aq, rows_bq, sga_q, sgb_q)
            wb(k1, rows_aq, rows_bq, swo_q)
            wb_wait(k0, rows_ap, rows_bp, swo_p)

            @pl.when(i < st2 - 1)
            def _():
                gath(k0 + 2, rows_ap, rows_bp, sga_p, sgb_p)

        wb_wait(steps - 1, rows_aq, rows_bq, swo_q)

    return gather_kernel(token_table, idx2d)


def _flatten_body(group, in_ref, out_ref):
    n_rows, seq = in_ref.shape
    for g in range(0, n_rows, group):
        flat = jnp.concatenate([in_ref[g + j, :] for j in range(group)])
        out_ref[pl.ds(g * seq, group * seq)] = flat


def _tc_flatten(x2d):
    batch, seq = x2d.shape
    block = 512
    group = 16
    return pl.pallas_call(
        functools.partial(_flatten_body, group),
        grid=(batch // block,),
        in_specs=[pl.BlockSpec((block, seq), lambda i: (i, 0))],
        out_specs=pl.BlockSpec((block * seq,), lambda i: (i,)),
        out_shape=jax.ShapeDtypeStruct((batch * seq,), jnp.int32),
    )(x2d)


def _ln_body(seq, d, h_ref, pos_ref, b_ref, out_ref):
    rows = pos_ref.shape[0]
    bvec = b_ref[...]
    sign = jnp.where(
        lax.broadcasted_iota(jnp.int32, (rows, 2 * d), 1) < d, 1.0, -1.0
    )
    for p in range(_SLABS_PER_BLOCK * seq // rows):
        r0 = p * rows
        h = h_ref[pl.ds(r0, rows), :] + pos_ref[...]
        hs = h * sign
        hh = h * h
        hhs = hs * h
        s_all = jnp.sum(h, axis=-1, keepdims=True)
        s_sgn = jnp.sum(hs, axis=-1, keepdims=True)
        q_all = jnp.sum(hh, axis=-1, keepdims=True)
        q_sgn = jnp.sum(hhs, axis=-1, keepdims=True)
        mean = (s_all + sign * s_sgn) * (0.5 / d)
        qh = (q_all + sign * q_sgn) * 0.5
        var_sum = qh - mean * mean * d
        scale = lax.rsqrt(var_sum * (1.0 / (d - 1)) + _EPS * _EPS)
        out = (h - mean) * scale + bvec
        out_ref[0, pl.ds(r0, rows), :] = out[:, :d]
        out_ref[1, pl.ds(r0, rows), :] = out[:, d:]


def _tc_layernorm(h_pair, pos_pair, b_pair, batch, seq, d):
    block_rows = seq * _SLABS_PER_BLOCK
    n_half = h_pair.shape[0]
    grid = n_half // block_rows
    half_span = n_half // 32
    bph = half_span // block_rows
    slab = pos_pair.shape[0]
    return pl.pallas_call(
        functools.partial(_ln_body, seq, d),
        grid=(grid,),
        in_specs=[
            pl.BlockSpec((block_rows, 2 * d), lambda i: (i, 0)),
            pl.BlockSpec((slab, 2 * d), lambda i: (0, 0)),
            pl.BlockSpec((1, 2 * d), lambda i: (0, 0)),
        ],
        out_specs=pl.BlockSpec(
            (2, block_rows, d), lambda i: (i // bph, i % bph, 0)
        ),
        out_shape=jax.ShapeDtypeStruct(
            (2 * n_half // half_span, half_span, d), jnp.float32),
    )(h_pair, pos_pair, b_pair)


def kernel(x, token_table, pos_table, a, b):
    batch, seq = x.shape
    d = token_table.shape[1]
    n = batch * seq
    idx2d = x.astype(jnp.int32).reshape(-1)
    gathered = _sc_gather(token_table, idx2d, seq)
    pos = pos_table[:seq]
    pos_pair = jnp.tile(jnp.concatenate([pos, pos], axis=1), (4, 1))
    b_pair = jnp.concatenate([b, b]).reshape(1, 2 * d)
    out = _tc_layernorm(gathered, pos_pair, b_pair, batch, seq, d)
    return out.reshape(batch, seq, d)
